# baseline (speedup 1.0000x reference)
import functools

import jax
import jax.numpy as jnp
from jax import lax
from jax.experimental import pallas as pl
from jax.experimental.pallas import tpu as pltpu
from jax.experimental.pallas import tpu_sc as plsc

N = 10000
E = 320000
D = 128

NC = 2
NS = 16
NW = NC * NS
K = 128
EPT = 10240
EP = NW * EPT
CHUNKS = EPT // K
CPP = 40
FAST_C = 1
FCH = 160
SCH = 0
ACC_ROWS = 10240
RPT = ACC_ROWS // NS


def _mesh():
    return plsc.VectorSubcoreMesh(
        core_axis_name="c", subcore_axis_name="s", num_cores=NC, num_subcores=NS)


def _sc_segment_sum():
    @functools.partial(
        pl.kernel,
        out_type=[jax.ShapeDtypeStruct((NC, ACC_ROWS, D), jnp.float32)],
        mesh=_mesh(),
        scratch_types=dict(
            acc=pltpu.VMEM_SHARED((ACC_ROWS, D), jnp.float32),
            srcv=pltpu.VMEM((CPP, K), jnp.int32),
            dstv=pltpu.VMEM((CPP, K), jnp.int32),
            rows=pltpu.VMEM((2, K, D), jnp.float32),
            gsem=pltpu.SemaphoreType.DMA,
            ssem=pltpu.SemaphoreType.DMA((2,)),
        ))
    def body(table_hbm, src_hbm, dst_hbm, zf_hbm, s_out,
             *, acc, srcv, dstv, rows, gsem, ssem):
        c = lax.axis_index("c")
        s = lax.axis_index("s")
        wid = s * NC + c

        pltpu.sync_copy(zf_hbm, acc.at[pl.ds(RPT * s, RPT)])
        plsc.subcore_barrier()

        def gather(i, p):
            pltpu.async_copy(table_hbm.at[srcv.at[i]], rows.at[p], gsem)

        def gwait(i, p):
            pltpu.make_async_copy(table_hbm.at[srcv.at[i]], rows.at[p],
                                  gsem).wait()

        def scat(i, p):
            pltpu.async_copy(rows.at[p], acc.at[dstv.at[i]], ssem.at[p],
                             add=True)

        def swait(i, p):
            pltpu.make_async_copy(rows.at[p], acc.at[dstv.at[i]],
                                  ssem.at[p]).wait()

        def ebody(i, carry):
            p = lax.rem(i, 2)
            gwait(i, p)

            @pl.when(i < CPP - 1)
            def _():
                @pl.when(i >= 1)
                def _():
                    swait(i - 1, 1 - p)
                gather(i + 1, 1 - p)

            scat(i, p)
            return carry

        base_chunk = jnp.where(c == FAST_C, s * FCH, NS * FCH + s * SCH)
        my_phases = jnp.where(c == FAST_C, FCH // CPP, SCH // CPP)

        for ph in range(FCH // CPP):
            @pl.when(ph < my_phases)
            def _():
                c0 = base_chunk + ph * CPP
                pltpu.sync_copy(src_hbm.at[pl.ds(c0, CPP)], srcv)
                pltpu.sync_copy(dst_hbm.at[pl.ds(c0, CPP)], dstv)
                gather(0, 0)
                lax.fori_loop(0, CPP, ebody, 0)
                swait(CPP - 2, 0)
                swait(CPP - 1, 1)
        plsc.subcore_barrier()

        pltpu.sync_copy(acc.at[pl.ds(RPT * s, RPT)], s_out.at[c, pl.ds(RPT * s, RPT)])

    return body


def _sc_count():
    @functools.partial(
        pl.kernel,
        out_type=[jax.ShapeDtypeStruct((NC, ACC_ROWS, D), jnp.float32)],
        mesh=_mesh(),
        scratch_types=dict(
            acc=pltpu.VMEM_SHARED((ACC_ROWS, D), jnp.float32),
            dstv=pltpu.VMEM((CPP, K), jnp.int32),
            onesv=pltpu.VMEM((K, D), jnp.float32),
        ))
    def body(dst_hbm, zf_hbm, c_out, *, acc, dstv, onesv):
        c = lax.axis_index("c")
        s = lax.axis_index("s")
        wid = s * NC + c
        one16 = jnp.ones((16,), jnp.float32)

        def fill(i, carry):
            for j in range(D // 16):
                onesv[i, pl.ds(j * 16, 16)] = one16
            return carry

        lax.fori_loop(0, K, fill, 0)
        pltpu.sync_copy(zf_hbm, acc.at[pl.ds(RPT * s, RPT)])
        plsc.subcore_barrier()

        def ebody(i, carry):
            pltpu.sync_copy(onesv, acc.at[dstv.at[i]], add=True)
            return carry

        for p in range(CHUNKS // CPP):
            pltpu.sync_copy(dst_hbm.at[pl.ds(wid * CHUNKS + p * CPP, CPP)], dstv)
            lax.fori_loop(0, CPP, ebody, 0)
        plsc.subcore_barrier()
        pltpu.sync_copy(acc.at[pl.ds(RPT * s, RPT)], c_out.at[c, pl.ds(RPT * s, RPT)])

    return body


def _tc_combine(do_relu: bool):
    R = 2000

    def body(s_ref, c_ref, f_ref, wl_ref, b_ref, wr_ref, o_ref):
        ssum = s_ref[0] + s_ref[1]
        cnt = c_ref[0, :, 0:1] + c_ref[1, :, 0:1]
        mean = ssum / jnp.maximum(cnt, 1.0)
        acc = (jnp.dot(mean, wl_ref[...], preferred_element_type=jnp.float32)
               + jnp.dot(f_ref[...], wr_ref[...], preferred_element_type=jnp.float32)
               + b_ref[...])
        if do_relu:
            acc = jnp.maximum(acc, 0.0)
        o_ref[...] = acc

    return pl.pallas_call(
        body,
        grid=(N // R,),
        in_specs=[
            pl.BlockSpec((2, R, D), lambda i: (0, i, 0)),
            pl.BlockSpec((2, R, D), lambda i: (0, i, 0)),
            pl.BlockSpec((R, D), lambda i: (i, 0)),
            pl.BlockSpec((D, D), lambda i: (0, 0)),
            pl.BlockSpec((1, D), lambda i: (0, 0)),
            pl.BlockSpec((D, D), lambda i: (0, 0)),
        ],
        out_specs=pl.BlockSpec((R, D), lambda i: (i, 0)),
        out_shape=jax.ShapeDtypeStruct((N, D), jnp.float32),
    )


def kernel(x, edge_index, W1_l, b1_l, W1_r, W2_l, b2_l, W2_r):
    src = edge_index[0]
    dst = edge_index[1]
    pad = EP - E
    srcp = jnp.concatenate([src, jnp.zeros((pad,), jnp.int32)]).reshape(EP // K, K)
    dstp = jnp.concatenate([dst, jnp.full((pad,), N, jnp.int32)]).reshape(EP // K, K)

    zf = jnp.zeros((RPT, D), jnp.float32)

    (cnt,) = _sc_count()(dstp, zf)
    (s1,) = _sc_segment_sum()(x, srcp, dstp, zf)

    h = _tc_combine(True)(s1, cnt, x, W1_l.T, b1_l.reshape(1, D), W1_r.T)

    (s2,) = _sc_segment_sum()(h, srcp, dstp, zf)

    out = _tc_combine(False)(s2, cnt, h, W2_l.T, b2_l.reshape(1, D), W2_r.T)
    return out


# speedup vs baseline: 3.1753x; 3.1753x over previous
import functools

import jax
import jax.numpy as jnp
from jax import lax
from jax.experimental import pallas as pl
from jax.experimental.pallas import tpu as pltpu
from jax.experimental.pallas import tpu_sc as plsc

N = 10000
E = 320000
D = 128

NC = 2
NS = 16
NW = NC * NS
K = 128
EPT = 10240
EP = NW * EPT
CHUNKS = EPT // K
CPP = 40
FCH = 80
SCH = 80
ACC_ROWS = 10240
RPT = ACC_ROWS // NS


def _mesh():
    return plsc.VectorSubcoreMesh(
        core_axis_name="c", subcore_axis_name="s", num_cores=NC, num_subcores=NS)


def _sc_segment_sum():
    @functools.partial(
        pl.kernel,
        out_type=[jax.ShapeDtypeStruct((NC, ACC_ROWS, D), jnp.float32)],
        mesh=_mesh(),
        scratch_types=dict(
            acc=pltpu.VMEM_SHARED((ACC_ROWS, D), jnp.float32),
            srcv=pltpu.VMEM((CPP, K), jnp.int32),
            dstv=pltpu.VMEM((CPP, K), jnp.int32),
            rows=pltpu.VMEM((2, K, D), jnp.float32),
            gsem=pltpu.SemaphoreType.DMA,
            ssem=pltpu.SemaphoreType.DMA((2,)),
        ))
    def body(table_hbm, src_hbm, dst_hbm, zf_hbm, s_out,
             *, acc, srcv, dstv, rows, gsem, ssem):
        c = lax.axis_index("c")
        s = lax.axis_index("s")
        wid = s * NC + c

        pltpu.sync_copy(zf_hbm, acc.at[pl.ds(RPT * s, RPT)])
        plsc.subcore_barrier()

        def gather(i, p):
            pltpu.async_copy(table_hbm.at[srcv.at[i]], rows.at[p], gsem)

        def gwait(i, p):
            pltpu.make_async_copy(table_hbm.at[srcv.at[i]], rows.at[p],
                                  gsem).wait()

        def scat(i, p):
            pltpu.async_copy(rows.at[p], acc.at[dstv.at[i]], ssem.at[p],
                             add=True)

        def swait(i, p):
            pltpu.make_async_copy(rows.at[p], acc.at[dstv.at[i]],
                                  ssem.at[p]).wait()

        def ebody(i, carry):
            p = lax.rem(i, 2)
            gwait(i, p)

            @pl.when(i < CPP - 1)
            def _():
                @pl.when(i >= 1)
                def _():
                    swait(i - 1, 1 - p)
                gather(i + 1, 1 - p)

            scat(i, p)
            return carry

        base_chunk = jnp.where(c == 1, s * FCH, NS * FCH + s * SCH)

        for ph in range(FCH // CPP):
            c0 = base_chunk + ph * CPP
            pltpu.sync_copy(src_hbm.at[pl.ds(c0, CPP)], srcv)
            pltpu.sync_copy(dst_hbm.at[pl.ds(c0, CPP)], dstv)
            gather(0, 0)
            lax.fori_loop(0, CPP, ebody, 0)
            swait(CPP - 2, 0)
            swait(CPP - 1, 1)
        plsc.subcore_barrier()

        pltpu.sync_copy(acc.at[pl.ds(RPT * s, RPT)], s_out.at[c, pl.ds(RPT * s, RPT)])

    return body


def _sc_count():
    @functools.partial(
        pl.kernel,
        out_type=[jax.ShapeDtypeStruct((NC, ACC_ROWS, D), jnp.float32)],
        mesh=_mesh(),
        scratch_types=dict(
            acc=pltpu.VMEM_SHARED((ACC_ROWS, D), jnp.float32),
            dstv=pltpu.VMEM((CPP, K), jnp.int32),
            onesv=pltpu.VMEM((K, D), jnp.float32),
        ))
    def body(dst_hbm, zf_hbm, c_out, *, acc, dstv, onesv):
        c = lax.axis_index("c")
        s = lax.axis_index("s")
        wid = s * NC + c
        one16 = jnp.ones((16,), jnp.float32)

        def fill(i, carry):
            for j in range(D // 16):
                onesv[i, pl.ds(j * 16, 16)] = one16
            return carry

        lax.fori_loop(0, K, fill, 0)
        pltpu.sync_copy(zf_hbm, acc.at[pl.ds(RPT * s, RPT)])
        plsc.subcore_barrier()

        def ebody(i, carry):
            pltpu.sync_copy(onesv, acc.at[dstv.at[i]], add=True)
            return carry

        for p in range(CHUNKS // CPP):
            pltpu.sync_copy(dst_hbm.at[pl.ds(wid * CHUNKS + p * CPP, CPP)], dstv)
            lax.fori_loop(0, CPP, ebody, 0)
        plsc.subcore_barrier()
        pltpu.sync_copy(acc.at[pl.ds(RPT * s, RPT)], c_out.at[c, pl.ds(RPT * s, RPT)])

    return body


def _tc_combine(do_relu: bool):
    R = 2000

    def body(s_ref, c_ref, f_ref, wl_ref, b_ref, wr_ref, o_ref):
        ssum = s_ref[0] + s_ref[1]
        cnt = c_ref[0, :, 0:1] + c_ref[1, :, 0:1]
        mean = ssum / jnp.maximum(cnt, 1.0)
        acc = (jnp.dot(mean, wl_ref[...], preferred_element_type=jnp.float32)
               + jnp.dot(f_ref[...], wr_ref[...], preferred_element_type=jnp.float32)
               + b_ref[...])
        if do_relu:
            acc = jnp.maximum(acc, 0.0)
        o_ref[...] = acc

    return pl.pallas_call(
        body,
        grid=(N // R,),
        in_specs=[
            pl.BlockSpec((2, R, D), lambda i: (0, i, 0)),
            pl.BlockSpec((2, R, D), lambda i: (0, i, 0)),
            pl.BlockSpec((R, D), lambda i: (i, 0)),
            pl.BlockSpec((D, D), lambda i: (0, 0)),
            pl.BlockSpec((1, D), lambda i: (0, 0)),
            pl.BlockSpec((D, D), lambda i: (0, 0)),
        ],
        out_specs=pl.BlockSpec((R, D), lambda i: (i, 0)),
        out_shape=jax.ShapeDtypeStruct((N, D), jnp.float32),
    )


def kernel(x, edge_index, W1_l, b1_l, W1_r, W2_l, b2_l, W2_r):
    src = edge_index[0]
    dst = edge_index[1]
    pad = EP - E
    pr = jnp.arange(pad, dtype=jnp.int32)
    srcp = jnp.concatenate([src, pr % N]).reshape(EP // K, K)
    dstp = jnp.concatenate([dst, N + pr % (ACC_ROWS - N)]).reshape(EP // K, K)

    zf = jnp.zeros((RPT, D), jnp.float32)

    (cnt,) = _sc_count()(dstp, zf)
    (s1,) = _sc_segment_sum()(x, srcp, dstp, zf)

    h = _tc_combine(True)(s1, cnt, x, W1_l.T, b1_l.reshape(1, D), W1_r.T)

    (s2,) = _sc_segment_sum()(h, srcp, dstp, zf)

    out = _tc_combine(False)(s2, cnt, h, W2_l.T, b2_l.reshape(1, D), W2_r.T)
    return out


# speedup vs baseline: 3.1840x; 1.0027x over previous
import functools

import jax
import jax.numpy as jnp
from jax import lax
from jax.experimental import pallas as pl
from jax.experimental.pallas import tpu as pltpu
from jax.experimental.pallas import tpu_sc as plsc

N = 10000
E = 320000
D = 128

NC = 2
NS = 16
NW = NC * NS
K = 128
EPT = 10240
EP = NW * EPT
CHUNKS = EPT // K
CPP = 40
FCH = 80
SCH = 80
ACC_ROWS = 10240
RPT = ACC_ROWS // NS


def _mesh():
    return plsc.VectorSubcoreMesh(
        core_axis_name="c", subcore_axis_name="s", num_cores=NC, num_subcores=NS)


def _sc_segment_sum():
    @functools.partial(
        pl.kernel,
        out_type=[jax.ShapeDtypeStruct((NC, ACC_ROWS, D), jnp.float32)],
        mesh=_mesh(),
        scratch_types=dict(
            acc=pltpu.VMEM_SHARED((ACC_ROWS, D), jnp.float32),
            srcv=pltpu.VMEM((CPP, K), jnp.int32),
            dstv=pltpu.VMEM((CPP, K), jnp.int32),
            rows=pltpu.VMEM((2, K, D), jnp.float32),
            gsem=pltpu.SemaphoreType.DMA,
            ssem=pltpu.SemaphoreType.DMA((2,)),
        ))
    def body(table_hbm, src_hbm, dst_hbm, zf_hbm, s_out,
             *, acc, srcv, dstv, rows, gsem, ssem):
        c = lax.axis_index("c")
        s = lax.axis_index("s")
        wid = s * NC + c

        pltpu.sync_copy(zf_hbm, acc.at[pl.ds(RPT * s, RPT)])
        plsc.subcore_barrier()

        def gather(i, p):
            pltpu.async_copy(table_hbm.at[srcv.at[i]], rows.at[p], gsem)

        def gwait(i, p):
            pltpu.make_async_copy(table_hbm.at[srcv.at[i]], rows.at[p],
                                  gsem).wait()

        def scat(i, p):
            pltpu.async_copy(rows.at[p], acc.at[dstv.at[i]], ssem.at[p],
                             add=True)

        def swait(i, p):
            pltpu.make_async_copy(rows.at[p], acc.at[dstv.at[i]],
                                  ssem.at[p]).wait()

        def ebody(i, carry):
            p = lax.rem(i, 2)
            gwait(i, p)

            @pl.when(i < CPP - 1)
            def _():
                @pl.when(i >= 1)
                def _():
                    swait(i - 1, 1 - p)
                gather(i + 1, 1 - p)

            scat(i, p)
            return carry

        base_chunk = jnp.where(c == 1, s * FCH, NS * FCH + s * SCH)

        for ph in range(FCH // CPP):
            c0 = base_chunk + ph * CPP
            pltpu.sync_copy(src_hbm.at[pl.ds(c0, CPP)], srcv)
            pltpu.sync_copy(dst_hbm.at[pl.ds(c0, CPP)], dstv)
            gather(0, 0)
            lax.fori_loop(0, CPP, ebody, 0)
            swait(CPP - 2, 0)
            swait(CPP - 1, 1)
        plsc.subcore_barrier()

        pltpu.sync_copy(acc.at[pl.ds(RPT * s, RPT)], s_out.at[c, pl.ds(RPT * s, RPT)])

    return body


def _sc_count():
    @functools.partial(
        pl.kernel,
        out_type=[jax.ShapeDtypeStruct((NC, ACC_ROWS, D), jnp.float32)],
        mesh=_mesh(),
        scratch_types=dict(
            acc=pltpu.VMEM_SHARED((ACC_ROWS, D), jnp.float32),
            dstv=pltpu.VMEM((CPP, K), jnp.int32),
            onesv=pltpu.VMEM((K, D), jnp.float32),
            sem=pltpu.SemaphoreType.DMA,
        ))
    def body(dst_hbm, zf_hbm, c_out, *, acc, dstv, onesv, sem):
        c = lax.axis_index("c")
        s = lax.axis_index("s")
        wid = s * NC + c
        one16 = jnp.ones((16,), jnp.float32)

        def fill(i, carry):
            for j in range(D // 16):
                onesv[i, pl.ds(j * 16, 16)] = one16
            return carry

        lax.fori_loop(0, K, fill, 0)
        pltpu.sync_copy(zf_hbm, acc.at[pl.ds(RPT * s, RPT)])
        plsc.subcore_barrier()

        def ebody(i, carry):
            pltpu.async_copy(onesv, acc.at[dstv.at[i]], sem, add=True)
            return carry

        def edrain(i, carry):
            pltpu.make_async_copy(onesv, acc.at[dstv.at[i]], sem).wait()
            return carry

        for p in range(CHUNKS // CPP):
            pltpu.sync_copy(dst_hbm.at[pl.ds(wid * CHUNKS + p * CPP, CPP)], dstv)
            lax.fori_loop(0, CPP, ebody, 0)
            lax.fori_loop(0, CPP, edrain, 0)
        plsc.subcore_barrier()
        pltpu.sync_copy(acc.at[pl.ds(RPT * s, RPT)], c_out.at[c, pl.ds(RPT * s, RPT)])

    return body


def _tc_combine(do_relu: bool):
    R = 2000

    def body(s_ref, c_ref, f_ref, wl_ref, b_ref, wr_ref, o_ref):
        ssum = s_ref[0] + s_ref[1]
        cnt = c_ref[0, :, 0:1] + c_ref[1, :, 0:1]
        mean = ssum / jnp.maximum(cnt, 1.0)
        acc = (jnp.dot(mean, wl_ref[...], preferred_element_type=jnp.float32)
               + jnp.dot(f_ref[...], wr_ref[...], preferred_element_type=jnp.float32)
               + b_ref[...])
        if do_relu:
            acc = jnp.maximum(acc, 0.0)
        o_ref[...] = acc

    return pl.pallas_call(
        body,
        grid=(N // R,),
        in_specs=[
            pl.BlockSpec((2, R, D), lambda i: (0, i, 0)),
            pl.BlockSpec((2, R, D), lambda i: (0, i, 0)),
            pl.BlockSpec((R, D), lambda i: (i, 0)),
            pl.BlockSpec((D, D), lambda i: (0, 0)),
            pl.BlockSpec((1, D), lambda i: (0, 0)),
            pl.BlockSpec((D, D), lambda i: (0, 0)),
        ],
        out_specs=pl.BlockSpec((R, D), lambda i: (i, 0)),
        out_shape=jax.ShapeDtypeStruct((N, D), jnp.float32),
    )


def kernel(x, edge_index, W1_l, b1_l, W1_r, W2_l, b2_l, W2_r):
    src = edge_index[0]
    dst = edge_index[1]
    pad = EP - E
    pr = jnp.arange(pad, dtype=jnp.int32)
    srcp = jnp.concatenate([src, pr % N]).reshape(EP // K, K)
    dstp = jnp.concatenate([dst, N + pr % (ACC_ROWS - N)]).reshape(EP // K, K)

    zf = jnp.zeros((RPT, D), jnp.float32)

    (cnt,) = _sc_count()(dstp, zf)
    (s1,) = _sc_segment_sum()(x, srcp, dstp, zf)

    h = _tc_combine(True)(s1, cnt, x, W1_l.T, b1_l.reshape(1, D), W1_r.T)

    (s2,) = _sc_segment_sum()(h, srcp, dstp, zf)

    out = _tc_combine(False)(s2, cnt, h, W2_l.T, b2_l.reshape(1, D), W2_r.T)
    return out


# speedup vs baseline: 3.1954x; 1.0036x over previous
import functools

import jax
import jax.numpy as jnp
from jax import lax
from jax.experimental import pallas as pl
from jax.experimental.pallas import tpu as pltpu
from jax.experimental.pallas import tpu_sc as plsc

N = 10000
E = 320000
D = 128

NC = 2
NS = 16
NW = NC * NS
K = 128
EPT = 10240
EP = NW * EPT
CHUNKS = EPT // K
CPP = 40
FCH = 80
SCH = 80
ACC_ROWS = 10240
RPT = ACC_ROWS // NS


def _mesh():
    return plsc.VectorSubcoreMesh(
        core_axis_name="c", subcore_axis_name="s", num_cores=NC, num_subcores=NS)


def _sc_segment_sum():
    @functools.partial(
        pl.kernel,
        out_type=[jax.ShapeDtypeStruct((NC, ACC_ROWS, D), jnp.float32)],
        mesh=_mesh(),
        scratch_types=dict(
            acc=pltpu.VMEM_SHARED((ACC_ROWS, D), jnp.float32),
            srcv=pltpu.VMEM((CPP, K), jnp.int32),
            dstv=pltpu.VMEM((CPP, K), jnp.int32),
            rows=pltpu.VMEM((2, K, D), jnp.float32),
            gsem=pltpu.SemaphoreType.DMA,
            ssem=pltpu.SemaphoreType.DMA((2,)),
        ))
    def body(table_hbm, src_hbm, dst_hbm, zf_hbm, s_out,
             *, acc, srcv, dstv, rows, gsem, ssem):
        c = lax.axis_index("c")
        s = lax.axis_index("s")

        pltpu.sync_copy(zf_hbm, acc.at[pl.ds(RPT * s, RPT)])
        plsc.subcore_barrier()

        def gather(i, p):
            pltpu.async_copy(table_hbm.at[srcv.at[i]], rows.at[p], gsem)

        def gwait(i, p):
            pltpu.make_async_copy(table_hbm.at[srcv.at[i]], rows.at[p],
                                  gsem).wait()

        def scat(i, p):
            pltpu.async_copy(rows.at[p], acc.at[dstv.at[i]], ssem.at[p],
                             add=True)

        def swait(i, p):
            pltpu.make_async_copy(rows.at[p], acc.at[dstv.at[i]],
                                  ssem.at[p]).wait()

        def ebody(i, carry):
            p = lax.rem(i, 2)
            gwait(i, p)

            @pl.when(i < CPP - 1)
            def _():
                @pl.when(i >= 1)
                def _():
                    swait(i - 1, 1 - p)
                gather(i + 1, 1 - p)

            scat(i, p)
            return carry

        base_chunk = jnp.where(c == 1, s * FCH, NS * FCH + s * SCH)

        for ph in range(FCH // CPP):
            c0 = base_chunk + ph * CPP
            pltpu.sync_copy(src_hbm.at[pl.ds(c0, CPP)], srcv)
            pltpu.sync_copy(dst_hbm.at[pl.ds(c0, CPP)], dstv)
            gather(0, 0)
            lax.fori_loop(0, CPP, ebody, 0)
            swait(CPP - 2, 0)
            swait(CPP - 1, 1)
        plsc.subcore_barrier()

        pltpu.sync_copy(acc.at[pl.ds(RPT * s, RPT)], s_out.at[c, pl.ds(RPT * s, RPT)])

    return body


def _sc_count():
    @functools.partial(
        pl.kernel,
        out_type=[jax.ShapeDtypeStruct((NC, ACC_ROWS, D), jnp.float32)],
        mesh=_mesh(),
        scratch_types=dict(
            acc=pltpu.VMEM_SHARED((ACC_ROWS, D), jnp.float32),
            dstv=pltpu.VMEM((CPP, K), jnp.int32),
            onesv=pltpu.VMEM((K, D), jnp.float32),
            sem=pltpu.SemaphoreType.DMA,
        ))
    def body(dst_hbm, zf_hbm, c_out, *, acc, dstv, onesv, sem):
        c = lax.axis_index("c")
        s = lax.axis_index("s")
        wid = s * NC + c
        one16 = jnp.ones((16,), jnp.float32)

        def fill(i, carry):
            for j in range(D // 16):
                onesv[i, pl.ds(j * 16, 16)] = one16
            return carry

        lax.fori_loop(0, K, fill, 0)
        pltpu.sync_copy(zf_hbm, acc.at[pl.ds(RPT * s, RPT)])
        plsc.subcore_barrier()

        def ebody(i, carry):
            pltpu.async_copy(onesv, acc.at[dstv.at[i]], sem, add=True)
            return carry

        def edrain(i, carry):
            pltpu.make_async_copy(onesv, acc.at[dstv.at[i]], sem).wait()
            return carry

        for p in range(CHUNKS // CPP):
            pltpu.sync_copy(dst_hbm.at[pl.ds(wid * CHUNKS + p * CPP, CPP)], dstv)
            lax.fori_loop(0, CPP, ebody, 0)
            lax.fori_loop(0, CPP, edrain, 0)
        plsc.subcore_barrier()
        pltpu.sync_copy(acc.at[pl.ds(RPT * s, RPT)], c_out.at[c, pl.ds(RPT * s, RPT)])

    return body


def _tc_combine(do_relu: bool):
    R = 2000

    def body(s_ref, c_ref, f_ref, wl_ref, b_ref, wr_ref, o_ref):
        ssum = s_ref[0] + s_ref[1]
        cnt = c_ref[0, :, 0:1] + c_ref[1, :, 0:1]
        mean = ssum / jnp.maximum(cnt, 1.0)
        acc = (jnp.dot(mean, wl_ref[...], preferred_element_type=jnp.float32)
               + jnp.dot(f_ref[...], wr_ref[...], preferred_element_type=jnp.float32)
               + b_ref[...])
        if do_relu:
            acc = jnp.maximum(acc, 0.0)
        o_ref[...] = acc

    return pl.pallas_call(
        body,
        grid=(N // R,),
        in_specs=[
            pl.BlockSpec((2, R, D), lambda i: (0, i, 0)),
            pl.BlockSpec((2, R, D), lambda i: (0, i, 0)),
            pl.BlockSpec((R, D), lambda i: (i, 0)),
            pl.BlockSpec((D, D), lambda i: (0, 0)),
            pl.BlockSpec((1, D), lambda i: (0, 0)),
            pl.BlockSpec((D, D), lambda i: (0, 0)),
        ],
        out_specs=pl.BlockSpec((R, D), lambda i: (i, 0)),
        out_shape=jax.ShapeDtypeStruct((N, D), jnp.float32),
    )


def kernel(x, edge_index, W1_l, b1_l, W1_r, W2_l, b2_l, W2_r):
    src = edge_index[0]
    dst = edge_index[1]
    pad = EP - E
    pr = jnp.arange(pad, dtype=jnp.int32)
    srcp = jnp.concatenate([src, pr % N]).reshape(EP // K, K)
    dstp = jnp.concatenate([dst, N + pr % (ACC_ROWS - N)]).reshape(EP // K, K)

    zf = jnp.zeros((RPT, D), jnp.float32)

    (cnt,) = _sc_count()(dstp, zf)
    (s1,) = _sc_segment_sum()(x, srcp, dstp, zf)

    h = _tc_combine(True)(s1, cnt, x, W1_l.T, b1_l.reshape(1, D), W1_r.T)

    (s2,) = _sc_segment_sum()(h, srcp, dstp, zf)

    out = _tc_combine(False)(s2, cnt, h, W2_l.T, b2_l.reshape(1, D), W2_r.T)
    return out
